# baseline (speedup 1.0000x reference)
import functools

import jax
import jax.numpy as jnp
from jax import lax
from jax.experimental import pallas as pl
from jax.experimental.pallas import tpu as pltpu
from jax.experimental.pallas import tpu_sc as plsc

NC = 2
NS = 16
NW = NC * NS

S0 = 4096
S1 = 50
D = 128
CHUNK = S0 // NW
NBUF = 5
N_OUTER = S1 // NBUF


def _emb_body(idx_hbm, table_hbm, out_hbm, idx_v, *rest):
    rows = rest[0:NBUF]
    gsems = rest[NBUF:2 * NBUF]
    osems = rest[2 * NBUF:3 * NBUF]
    wid = lax.axis_index("s") * NC + lax.axis_index("c")
    pltpu.sync_copy(idx_hbm.at[:, pl.ds(wid * CHUNK, CHUNK)], idx_v)

    for b in range(NBUF):
        pltpu.async_copy(table_hbm.at[idx_v.at[b]], rows[b], gsems[b])

    col = pl.ds(wid * CHUNK, CHUNK)
    N_CHUNKS = S1

    def outer(i, carry):
        for b in range(NBUF):
            g = i * NBUF + b
            pltpu.make_async_copy(
                table_hbm.at[idx_v.at[g]], rows[b], gsems[b]).wait()
            pltpu.async_copy(rows[b], out_hbm.at[g, col], osems[b])

            p = g - 1
            pb = (b - 1) % NBUF
            cond = (i >= 1) if b == 0 else (i <= N_OUTER - 2)

            @pl.when(cond)
            def _():
                pltpu.make_async_copy(
                    rows[pb], out_hbm.at[p, col], osems[pb]).wait()
                pltpu.async_copy(
                    table_hbm.at[idx_v.at[p + NBUF]], rows[pb], gsems[pb])
        return carry

    lax.fori_loop(0, N_OUTER, outer, 0)

    for b in range(NBUF):
        g = N_CHUNKS - NBUF + b
        pltpu.make_async_copy(
            rows[b], out_hbm.at[g, col], osems[b]).wait()


_emb = functools.partial(
    pl.kernel,
    out_type=jax.ShapeDtypeStruct((S1, S0, D), jnp.float32),
    mesh=plsc.VectorSubcoreMesh(core_axis_name="c", subcore_axis_name="s"),
    compiler_params=pltpu.CompilerParams(use_tc_tiling_on_sc=True),
    scratch_types=(
        [pltpu.VMEM((S1, CHUNK), jnp.int32)]
        + [pltpu.VMEM((CHUNK, D), jnp.float32) for _ in range(NBUF)]
        + [pltpu.SemaphoreType.DMA for _ in range(2 * NBUF)]
    ),
)(_emb_body)


@jax.jit
def kernel(inputs, table):
    idx = inputs.astype(jnp.int32).T
    out_t = _emb(idx, table)
    return out_t.transpose(1, 0, 2)
